# baseline (speedup 1.0000x reference)
import functools

import jax
import jax.numpy as jnp
from jax import lax
from jax.experimental import pallas as pl
from jax.experimental.pallas import tpu as pltpu
from jax.experimental.pallas import tpu_sc as plsc

_NC = 2
_NS = 16
_LANES = 16
_CHUNK = 64
_NBUF = 2


@functools.lru_cache(maxsize=None)
def _build(B, L, D, T):
    W = _NC * _NS
    assert W == 2 * B and D == 256 and T % (2 * _CHUNK) == 0
    NPAIR = T // (2 * _CHUNK)
    CW0, CW1 = D // 2, D // 2 + 4
    ZROW = L
    SENT = jnp.int32(0x3FFFFFFF)

    mesh = plsc.VectorSubcoreMesh(
        core_axis_name="c", subcore_axis_name="s",
        num_cores=_NC, num_subcores=_NS)

    @functools.partial(
        pl.kernel,
        out_type=jax.ShapeDtypeStruct((B * T, D + 4), jnp.float32),
        mesh=mesh,
        compiler_params=pltpu.CompilerParams(needs_layout_passes=False),
        scratch_types=[
            pltpu.VMEM((L,), jnp.int32),
            pltpu.VMEM((L + _LANES,), jnp.int32),
            pltpu.VMEM((T,), jnp.int32),
            pltpu.VMEM((L + 8, D // 2), jnp.float32),
            pltpu.VMEM((_NBUF, _CHUNK, D // 2 + 4), jnp.float32),
            pltpu.VMEM((T * 4,), jnp.float32),
            pltpu.SemaphoreType.DMA,
            pltpu.SemaphoreType.DMA,
            pltpu.SemaphoreType.DMA((_NBUF,)),
        ],
    )
    def sc_expand(enc_hbm, dur_hbm, fr_hbm, out_hbm,
                  dur_v, cum_v, m_v, stg, gbuf, fbuf, s_sem, f_sem, w_sems):
        wid = lax.axis_index("s") * _NC + lax.axis_index("c")
        b = wid % B
        ch = wid // B

        lane = lax.iota(jnp.int32, _LANES)
        zv16 = jnp.zeros((_LANES,), jnp.float32)

        stg_src = enc_hbm.at[pl.ds(b * L, L), pl.ds(ch * (D // 2), D // 2)]
        stg_dst = stg.at[pl.ds(0, L)]
        pltpu.async_copy(stg_src, stg_dst, s_sem)

        @pl.when(ch == 1)
        def _():
            pltpu.async_copy(fr_hbm.at[b], fbuf, f_sem)

        pltpu.sync_copy(dur_hbm.at[b], dur_v)

        cum_v[pl.ds(L, _LANES)] = jnp.full((_LANES,), SENT, jnp.int32)

        def cs_body(j, run):
            x = dur_v[pl.ds(j * _LANES, _LANES)]
            s = plsc.cumsum(x) + run
            cum_v[pl.ds(j * _LANES, _LANES)] = s
            return jnp.max(s)

        mel_len = lax.fori_loop(0, L // _LANES, cs_body, jnp.int32(0))

        @plsc.parallel_loop(0, T // _LANES, unroll=4)
        def z_body(i):
            m_v[pl.ds(i * _LANES, _LANES)] = jnp.zeros((_LANES,), jnp.int32)

        @plsc.parallel_loop(0, L // _LANES, unroll=2)
        def sct_body(j):
            c16 = cum_v[pl.ds(j * _LANES, _LANES)]
            cnx = cum_v[pl.ds(j * _LANES + 1, _LANES)]
            keep = (c16 != cnx) & (c16 >= 0) & (c16 < T)
            vals = j * _LANES + lane + 1
            plsc.store_scatter(m_v, [c16], vals, mask=keep)

        def mx_body(i, run):
            v = m_v[pl.ds(i * _LANES, _LANES)]
            s = jnp.maximum(plsc.cummax(v), run)
            t16 = i * _LANES + lane
            g = jnp.where(t16 < mel_len, s, jnp.int32(ZROW))
            m_v[pl.ds(i * _LANES, _LANES)] = g
            return jnp.max(s)

        lax.fori_loop(0, T // _LANES, mx_body, jnp.int32(0))

        for k in range(D // 2 // _LANES):
            stg[ZROW, pl.ds(k * _LANES, _LANES)] = zv16

        rpat = lax.shift_right_logical(lane, 2)
        fcol = D // 2 + (lane & 3)

        def expand_rows(c, sl):
            @plsc.parallel_loop(0, _CHUNK // _LANES, unroll=4)
            def grp_body(g2):
                q16 = m_v[pl.ds(c * _CHUNK + g2 * _LANES, _LANES)]
                for r in range(_LANES):
                    q = q16[r]
                    row = g2 * _LANES + r
                    for k in range(D // 2 // _LANES):
                        gbuf[sl, row, pl.ds(k * _LANES, _LANES)] = (
                            stg[q, pl.ds(k * _LANES, _LANES)])

        def merge_frames(c, sl):
            @plsc.parallel_loop(0, _CHUNK * 4 // _LANES, unroll=2)
            def mg_body(i2):
                vals = fbuf[pl.ds(c * _CHUNK * 4 + i2 * _LANES, _LANES)]
                plsc.store_scatter(gbuf.at[sl], [i2 * 4 + rpat, fcol], vals)

        def run_side(co, width, with_frames):
            pltpu.make_async_copy(stg_src, stg_dst, s_sem).wait()
            if with_frames:
                pltpu.make_async_copy(fr_hbm.at[b], fbuf, f_sem).wait()

            def w_dst(c):
                return out_hbm.at[pl.ds(b * T + c * _CHUNK, _CHUNK),
                                  pl.ds(co, width)]

            def w_src(sl):
                return gbuf.at[sl, :, pl.ds(0, width)]

            def pair_body(i, _):
                for sl in range(_NBUF):
                    c = _NBUF * i + sl

                    @pl.when(i > 0)
                    def _():
                        pltpu.make_async_copy(
                            w_src(sl), w_dst(c - _NBUF), w_sems.at[sl]).wait()

                    expand_rows(c, sl)
                    if with_frames:
                        merge_frames(c, sl)
                    pltpu.async_copy(w_src(sl), w_dst(c), w_sems.at[sl])
                return 0

            lax.fori_loop(0, NPAIR, pair_body, 0)
            for sl in range(_NBUF):
                c = _NBUF * (NPAIR - 1) + sl
                pltpu.make_async_copy(
                    w_src(sl), w_dst(c), w_sems.at[sl]).wait()

        @pl.when(ch == 0)
        def _():
            run_side(0, CW0, False)

        @pl.when(ch == 1)
        def _():
            run_side(D // 2, CW1, True)

    return sc_expand


def kernel(encoder_outputs, durations, frames_positions, input_lengths):
    B, L, D = encoder_outputs.shape
    T, DP = frames_positions.shape[1], frames_positions.shape[2]
    enc = encoder_outputs.reshape(B * L, D)
    fr = frames_positions.reshape(B, T * DP)
    out = _build(B, L, D, T)(enc, durations, fr)
    return out.reshape(B, T, D + DP)


# speedup vs baseline: 1.2806x; 1.2806x over previous
import functools

import jax
import jax.numpy as jnp
from jax import lax
from jax.experimental import pallas as pl
from jax.experimental.pallas import tpu as pltpu
from jax.experimental.pallas import tpu_sc as plsc

_NC = 2
_NS = 16
_LANES = 16
_CHUNK = 64
_NBUF = 2


@functools.lru_cache(maxsize=None)
def _build(B, L, D, T):
    W = _NC * _NS
    assert W == 2 * B and D == 256 and T % (2 * _CHUNK) == 0
    NPAIR = T // (2 * _CHUNK)
    CW0, CW1 = D // 2, D // 2 + 4
    ZROW = L
    SENT = jnp.int32(0x3FFFFFFF)

    mesh = plsc.VectorSubcoreMesh(
        core_axis_name="c", subcore_axis_name="s",
        num_cores=_NC, num_subcores=_NS)

    @functools.partial(
        pl.kernel,
        out_type=jax.ShapeDtypeStruct((B * T, D + 4), jnp.float32),
        mesh=mesh,
        compiler_params=pltpu.CompilerParams(needs_layout_passes=False),
        scratch_types=[
            pltpu.VMEM((L,), jnp.int32),
            pltpu.VMEM((L + _LANES,), jnp.int32),
            pltpu.VMEM((T,), jnp.int32),
            pltpu.VMEM((L + 8, D // 2), jnp.float32),
            pltpu.VMEM((_NBUF, _CHUNK, D // 2 + 4), jnp.float32),
            pltpu.VMEM((T * 4,), jnp.float32),
            pltpu.SemaphoreType.DMA,
            pltpu.SemaphoreType.DMA,
            pltpu.SemaphoreType.DMA((_NBUF,)),
        ],
    )
    def sc_expand(enc_hbm, dur_hbm, fr_hbm, out_hbm,
                  dur_v, cum_v, m_v, stg, gbuf, fbuf, s_sem, f_sem, w_sems):
        wid = lax.axis_index("s") * _NC + lax.axis_index("c")
        b = wid % B
        ch = wid // B

        lane = lax.iota(jnp.int32, _LANES)
        zv16 = jnp.zeros((_LANES,), jnp.float32)

        stg_src = enc_hbm.at[pl.ds(b * L, L), pl.ds(ch * (D // 2), D // 2)]
        stg_dst = stg.at[pl.ds(0, L)]
        pltpu.async_copy(stg_src, stg_dst, s_sem)

        @pl.when(ch == 1)
        def _():
            pltpu.async_copy(fr_hbm.at[b], fbuf, f_sem)

        pltpu.sync_copy(dur_hbm.at[b], dur_v)

        cum_v[pl.ds(L, _LANES)] = jnp.full((_LANES,), SENT, jnp.int32)

        def cs_body(j, run):
            x = dur_v[pl.ds(j * _LANES, _LANES)]
            s = plsc.cumsum(x) + run
            cum_v[pl.ds(j * _LANES, _LANES)] = s
            return s[_LANES - 1]

        mel_len = lax.fori_loop(0, L // _LANES, cs_body, jnp.int32(0))

        @plsc.parallel_loop(0, T // _LANES, unroll=4)
        def z_body(i):
            m_v[pl.ds(i * _LANES, _LANES)] = jnp.zeros((_LANES,), jnp.int32)

        @plsc.parallel_loop(0, L // _LANES, unroll=2)
        def sct_body(j):
            c16 = cum_v[pl.ds(j * _LANES, _LANES)]
            cnx = cum_v[pl.ds(j * _LANES + 1, _LANES)]
            keep = (c16 != cnx) & (c16 >= 0) & (c16 < T)
            vals = j * _LANES + lane + 1
            plsc.store_scatter(m_v, [c16], vals, mask=keep)

        def mx_body(i, run):
            v = m_v[pl.ds(i * _LANES, _LANES)]
            s = jnp.maximum(plsc.cummax(v), run)
            t16 = i * _LANES + lane
            g = jnp.where(t16 < mel_len, s, jnp.int32(ZROW))
            m_v[pl.ds(i * _LANES, _LANES)] = g
            return s[_LANES - 1]

        lax.fori_loop(0, T // _LANES, mx_body, jnp.int32(0))

        for k in range(D // 2 // _LANES):
            stg[ZROW, pl.ds(k * _LANES, _LANES)] = zv16

        rpat = lax.shift_right_logical(lane, 2)
        fcol = D // 2 + (lane & 3)

        def expand_rows(c, sl):
            @plsc.parallel_loop(0, _CHUNK // _LANES, unroll=2)
            def grp_body(g2):
                q16 = m_v[pl.ds(c * _CHUNK + g2 * _LANES, _LANES)]
                for r in range(_LANES):
                    q = q16[r]
                    row = g2 * _LANES + r
                    for k in range(D // 2 // _LANES):
                        gbuf[sl, row, pl.ds(k * _LANES, _LANES)] = (
                            stg[q, pl.ds(k * _LANES, _LANES)])

        def merge_frames(c, sl):
            @plsc.parallel_loop(0, _CHUNK * 4 // _LANES, unroll=2)
            def mg_body(i2):
                vals = fbuf[pl.ds(c * _CHUNK * 4 + i2 * _LANES, _LANES)]
                plsc.store_scatter(gbuf.at[sl], [i2 * 4 + rpat, fcol], vals)

        def run_side(co, width, with_frames):
            pltpu.make_async_copy(stg_src, stg_dst, s_sem).wait()
            if with_frames:
                pltpu.make_async_copy(fr_hbm.at[b], fbuf, f_sem).wait()

            def w_dst(c):
                return out_hbm.at[pl.ds(b * T + c * _CHUNK, _CHUNK),
                                  pl.ds(co, width)]

            def w_src(sl):
                return gbuf.at[sl, :, pl.ds(0, width)]

            def pair_body(i, _):
                for sl in range(_NBUF):
                    c = _NBUF * i + sl

                    @pl.when(i > 0)
                    def _():
                        pltpu.make_async_copy(
                            w_src(sl), w_dst(c - _NBUF), w_sems.at[sl]).wait()

                    expand_rows(c, sl)
                    if with_frames:
                        merge_frames(c, sl)
                    pltpu.async_copy(w_src(sl), w_dst(c), w_sems.at[sl])
                return 0

            lax.fori_loop(0, NPAIR, pair_body, 0)
            for sl in range(_NBUF):
                c = _NBUF * (NPAIR - 1) + sl
                pltpu.make_async_copy(
                    w_src(sl), w_dst(c), w_sems.at[sl]).wait()

        @pl.when(ch == 0)
        def _():
            run_side(0, CW0, False)

        @pl.when(ch == 1)
        def _():
            run_side(D // 2, CW1, True)

    return sc_expand


def kernel(encoder_outputs, durations, frames_positions, input_lengths):
    B, L, D = encoder_outputs.shape
    T, DP = frames_positions.shape[1], frames_positions.shape[2]
    enc = encoder_outputs.reshape(B * L, D)
    fr = frames_positions.reshape(B, T * DP)
    out = _build(B, L, D, T)(enc, durations, fr)
    return out.reshape(B, T, D + DP)
